# baseline (speedup 1.0000x reference)
import functools

import jax
import jax.numpy as jnp
from jax.experimental import pallas as pl


def _input_net_kernel(x_ref, w1_ref, b1_ref, scale_ref, shift_ref, w2_ref, b2_ref, o_ref):
    h = jnp.dot(x_ref[...], w1_ref[...], preferred_element_type=jnp.float32)
    h = h * scale_ref[...] + shift_ref[...]
    h = jnp.tanh(h)
    h = jnp.dot(h, w2_ref[...], preferred_element_type=jnp.float32) + b2_ref[...]
    o_ref[...] = jnp.maximum(h, 0.0)


def _edge_mlp_kernel(xi_ref, xj_ref, wa_ref, wb_ref, bc1_ref, w2_ref, bc2_ref, o_ref):
    m = jnp.dot(xi_ref[...], wa_ref[...], preferred_element_type=jnp.float32)
    m += jnp.dot(xj_ref[...], wb_ref[...], preferred_element_type=jnp.float32)
    m = jnp.maximum(m + bc1_ref[...], 0.0)
    m = jnp.dot(m, w2_ref[...], preferred_element_type=jnp.float32) + bc2_ref[...]
    o_ref[...] = jnp.maximum(m, 0.0)


def _edge_score_kernel(hs_ref, hd_ref, wa_ref, wb_ref, be1_ref, w2_ref, be2_ref, o_ref):
    e = jnp.dot(hs_ref[...], wa_ref[...], preferred_element_type=jnp.float32)
    e += jnp.dot(hd_ref[...], wb_ref[...], preferred_element_type=jnp.float32)
    e = jnp.maximum(e + be1_ref[...], 0.0)
    e = jnp.dot(e, w2_ref[...], preferred_element_type=jnp.float32) + be2_ref[...]
    o_ref[...] = jax.nn.sigmoid(e)


def _full(shape):
    return pl.BlockSpec(shape, lambda i: tuple(0 for _ in shape))


def _input_net(x, W1, b1, bn_g, bn_b, bn_rm, bn_rv, W2, b2):
    n, d = x.shape
    h_dim = W1.shape[0]
    inv = bn_g / jnp.sqrt(bn_rv + 1e-5)
    scale = inv
    shift = (b1 - bn_rm) * inv + bn_b
    return pl.pallas_call(
        _input_net_kernel,
        grid=(1,),
        in_specs=[
            _full((n, d)),
            _full((d, h_dim)),
            _full((1, h_dim)),
            _full((1, h_dim)),
            _full((1, h_dim)),
            _full((h_dim, h_dim)),
            _full((1, h_dim)),
        ],
        out_specs=_full((n, h_dim)),
        out_shape=jax.ShapeDtypeStruct((n, h_dim), jnp.float32),
    )(x, W1.T, b1[None], scale[None], shift[None], W2.T, b2[None])


def _edge_mlp(xi, xj, WA, WB, bc1, W2T, bc2, blk):
    ep, h2 = xi.shape[0], WA.shape[1]
    h_dim = W2T.shape[1]
    grid = (ep // blk,)
    return pl.pallas_call(
        _edge_mlp_kernel,
        grid=grid,
        in_specs=[
            pl.BlockSpec((blk, xi.shape[1]), lambda i: (i, 0)),
            pl.BlockSpec((blk, xj.shape[1]), lambda i: (i, 0)),
            _full(WA.shape),
            _full(WB.shape),
            _full((1, h2)),
            _full(W2T.shape),
            _full((1, h_dim)),
        ],
        out_specs=pl.BlockSpec((blk, h_dim), lambda i: (i, 0)),
        out_shape=jax.ShapeDtypeStruct((ep, h_dim), jnp.float32),
    )(xi, xj, WA, WB, bc1[None], W2T, bc2[None])


def _edge_score(hs, hd, WA, WB, be1, W2T, be2, blk):
    ep = hs.shape[0]
    h_dim = WA.shape[1]
    grid = (ep // blk,)
    return pl.pallas_call(
        _edge_score_kernel,
        grid=grid,
        in_specs=[
            pl.BlockSpec((blk, hs.shape[1]), lambda i: (i, 0)),
            pl.BlockSpec((blk, hd.shape[1]), lambda i: (i, 0)),
            _full(WA.shape),
            _full(WB.shape),
            _full((1, h_dim)),
            _full(W2T.shape),
            _full((1, 1)),
        ],
        out_specs=pl.BlockSpec((blk, 1), lambda i: (i, 0)),
        out_shape=jax.ShapeDtypeStruct((ep, 1), jnp.float32),
    )(hs, hd, WA, WB, be1[None], W2T, be2[None])


def kernel(x, edge_index, W1, b1, bn_g, bn_b, bn_rm, bn_rv, W2, b2, Wc1, bc1, Wc2, bc2, We1, be1, We2, be2):
    n = x.shape[0]
    e_cnt = edge_index.shape[1]
    h_dim = W1.shape[0]

    loops = jnp.arange(n, dtype=edge_index.dtype)
    ei = jnp.concatenate([edge_index, jnp.stack([loops, loops])], axis=1)
    ei2 = jnp.concatenate([ei, ei[::-1]], axis=1)
    k = ei2[0] * n + ei2[1]
    ks = jnp.sort(k)
    mask = jnp.concatenate([jnp.ones((1,), dtype=bool), ks[1:] != ks[:-1]])
    row = ks // n
    col = ks % n

    ec = ks.shape[0]
    blk = 8192
    ec_pad = ((ec + blk - 1) // blk) * blk
    pad = ec_pad - ec
    rowp = jnp.concatenate([row, jnp.zeros((pad,), row.dtype)])
    colp = jnp.concatenate([col, jnp.zeros((pad,), col.dtype)])
    segp = jnp.concatenate([jnp.where(mask, row, n), jnp.full((pad,), n, row.dtype)])

    h = _input_net(x, W1, b1, bn_g, bn_b, bn_rm, bn_rv, W2, b2)

    A = Wc1[:, :h_dim]
    B = Wc1[:, h_dim:]
    WA = (A - B).T
    WB = B.T
    W2T = Wc2.T

    for _ in range(3):
        xi = jnp.take(h, rowp, axis=0)
        xj = jnp.take(h, colp, axis=0)
        m = _edge_mlp(xi, xj, WA, WB, bc1, W2T, bc2, blk)
        h = jax.ops.segment_sum(m, segp, num_segments=n)

    src = edge_index[0]
    dst = edge_index[1]
    eblk = 8192
    e_pad = ((e_cnt + eblk - 1) // eblk) * eblk
    srcp = jnp.concatenate([src, jnp.zeros((e_pad - e_cnt,), src.dtype)])
    dstp = jnp.concatenate([dst, jnp.zeros((e_pad - e_cnt,), dst.dtype)])
    hs = jnp.take(h, srcp, axis=0)
    hd = jnp.take(h, dstp, axis=0)
    EA = We1[:, :h_dim].T
    EB = We1[:, h_dim:].T
    scores = _edge_score(hs, hd, EA, EB, be1, We2.T, be2, eblk)
    return scores[:e_cnt, 0]


# speedup vs baseline: 2.7478x; 2.7478x over previous
import functools

import jax
import jax.numpy as jnp
from jax import lax
from jax.experimental import pallas as pl
from jax.experimental.pallas import tpu as pltpu
from jax.experimental.pallas import tpu_sc as plsc

NW = 32
NC = 2
ROWS_PER_TX = 128
TX_PER_CHUNK = 8


def _sc_mesh():
    return plsc.VectorSubcoreMesh(core_axis_name="c", subcore_axis_name="s")


def _make_gather2(n_pad, h_dim, ec_pad):
    chunk = ROWS_PER_TX * TX_PER_CHUNK
    per_w = ec_pad // NW
    n_chunks = per_w // chunk
    assert per_w % chunk == 0

    @functools.partial(
        pl.kernel,
        out_type=[jax.ShapeDtypeStruct((ec_pad, h_dim), jnp.float32),
                  jax.ShapeDtypeStruct((ec_pad, h_dim), jnp.float32)],
        mesh=_sc_mesh(),
        compiler_params=pltpu.CompilerParams(use_tc_tiling_on_sc=False),
        scratch_types=[
            pltpu.VMEM((TX_PER_CHUNK, ROWS_PER_TX), jnp.int32),
            pltpu.VMEM((TX_PER_CHUNK, ROWS_PER_TX), jnp.int32),
            pltpu.VMEM((chunk, h_dim), jnp.float32),
            pltpu.VMEM((chunk, h_dim), jnp.float32),
            pltpu.SemaphoreType.DMA,
        ],
    )
    def gather2(tab_hbm, rows_hbm, cols_hbm, xi_hbm, xj_hbm, ia, ib, ba, bb, sem):
        wid = lax.axis_index("s") * NC + lax.axis_index("c")
        base_r = wid * (per_w // ROWS_PER_TX)

        @pl.loop(0, n_chunks)
        def _(i):
            r0 = base_r + i * TX_PER_CHUNK
            off = r0 * ROWS_PER_TX
            pltpu.sync_copy(rows_hbm.at[pl.ds(r0, TX_PER_CHUNK)], ia)
            pltpu.sync_copy(cols_hbm.at[pl.ds(r0, TX_PER_CHUNK)], ib)
            copies = []
            for j in range(TX_PER_CHUNK):
                copies.append(pltpu.async_copy(
                    tab_hbm.at[ia.at[j]],
                    ba.at[pl.ds(j * ROWS_PER_TX, ROWS_PER_TX)], sem))
                copies.append(pltpu.async_copy(
                    tab_hbm.at[ib.at[j]],
                    bb.at[pl.ds(j * ROWS_PER_TX, ROWS_PER_TX)], sem))
            for cp in copies:
                cp.wait()
            pltpu.sync_copy(ba, xi_hbm.at[pl.ds(off, chunk)])
            pltpu.sync_copy(bb, xj_hbm.at[pl.ds(off, chunk)])

    return gather2


def _make_scatter_add(nt, h_dim, ec_pad):
    chunk = ROWS_PER_TX * TX_PER_CHUNK
    per_w = ec_pad // NW
    n_chunks = per_w // chunk
    rows_per_tile = nt // 16
    assert per_w % chunk == 0 and nt % 16 == 0

    @functools.partial(
        pl.kernel,
        out_type=jax.ShapeDtypeStruct((NC, nt, h_dim), jnp.float32),
        mesh=_sc_mesh(),
        compiler_params=pltpu.CompilerParams(use_tc_tiling_on_sc=False),
        scratch_types=[
            pltpu.VMEM((TX_PER_CHUNK, ROWS_PER_TX), jnp.int32),
            pltpu.VMEM((chunk, h_dim), jnp.float32),
            pltpu.VMEM_SHARED((nt, h_dim), jnp.float32),
        ],
    )
    def scatter_add(m_hbm, seg_hbm, zeros_hbm, out_hbm, idx, buf, acc):
        cid = lax.axis_index("c")
        sid = lax.axis_index("s")
        wid = sid * NC + cid
        base_r = wid * (per_w // ROWS_PER_TX)

        @pl.when(sid == 0)
        def _():
            pltpu.sync_copy(zeros_hbm, acc)

        plsc.subcore_barrier()

        @pl.loop(0, n_chunks)
        def _(i):
            r0 = base_r + i * TX_PER_CHUNK
            off = r0 * ROWS_PER_TX
            pltpu.sync_copy(seg_hbm.at[pl.ds(r0, TX_PER_CHUNK)], idx)
            pltpu.sync_copy(m_hbm.at[pl.ds(off, chunk)], buf)
            for j in range(TX_PER_CHUNK):
                pltpu.sync_copy(
                    buf.at[pl.ds(j * ROWS_PER_TX, ROWS_PER_TX)],
                    acc.at[idx.at[j]], add=True)

        plsc.subcore_barrier()
        pltpu.sync_copy(acc.at[pl.ds(sid * rows_per_tile, rows_per_tile)],
                        out_hbm.at[cid].at[pl.ds(sid * rows_per_tile, rows_per_tile)])

    return scatter_add


def _input_net_kernel(x_ref, w1_ref, scale_ref, shift_ref, w2_ref, b2_ref, o_ref):
    h = jnp.dot(x_ref[...], w1_ref[...], preferred_element_type=jnp.float32)
    h = h * scale_ref[...] + shift_ref[...]
    h = jnp.tanh(h)
    h = jnp.dot(h, w2_ref[...], preferred_element_type=jnp.float32) + b2_ref[...]
    o_ref[...] = jnp.maximum(h, 0.0)


def _edge_mlp_kernel(xi_ref, xj_ref, wa_ref, wb_ref, bc1_ref, w2_ref, bc2_ref, o_ref):
    m = jnp.dot(xi_ref[...], wa_ref[...], preferred_element_type=jnp.float32)
    m += jnp.dot(xj_ref[...], wb_ref[...], preferred_element_type=jnp.float32)
    m = jnp.maximum(m + bc1_ref[...], 0.0)
    m = jnp.dot(m, w2_ref[...], preferred_element_type=jnp.float32) + bc2_ref[...]
    o_ref[...] = jnp.maximum(m, 0.0)


def _edge_score_kernel(hs_ref, hd_ref, wa_ref, wb_ref, be1_ref, w2_ref, be2_ref, o_ref):
    e = jnp.dot(hs_ref[...], wa_ref[...], preferred_element_type=jnp.float32)
    e += jnp.dot(hd_ref[...], wb_ref[...], preferred_element_type=jnp.float32)
    e = jnp.maximum(e + be1_ref[...], 0.0)
    e = jnp.dot(e, w2_ref[...], preferred_element_type=jnp.float32) + be2_ref[...]
    o_ref[...] = jax.nn.sigmoid(e)


def _full(shape):
    return pl.BlockSpec(shape, lambda i: tuple(0 for _ in shape))


def _input_net(x, W1, b1, bn_g, bn_b, bn_rm, bn_rv, W2, b2):
    n, d = x.shape
    h_dim = W1.shape[0]
    inv = bn_g / jnp.sqrt(bn_rv + 1e-5)
    scale = inv
    shift = (b1 - bn_rm) * inv + bn_b
    return pl.pallas_call(
        _input_net_kernel,
        grid=(1,),
        in_specs=[_full((n, d)), _full((d, h_dim)), _full((1, h_dim)),
                  _full((1, h_dim)), _full((h_dim, h_dim)), _full((1, h_dim))],
        out_specs=_full((n, h_dim)),
        out_shape=jax.ShapeDtypeStruct((n, h_dim), jnp.float32),
    )(x, W1.T, scale[None], shift[None], W2.T, b2[None])


def _edge_mlp(xi, xj, WA, WB, bc1, W2T, bc2, blk):
    ep, h2 = xi.shape[0], WA.shape[1]
    h_dim = W2T.shape[1]
    return pl.pallas_call(
        _edge_mlp_kernel,
        grid=(ep // blk,),
        in_specs=[pl.BlockSpec((blk, xi.shape[1]), lambda i: (i, 0)),
                  pl.BlockSpec((blk, xj.shape[1]), lambda i: (i, 0)),
                  _full(WA.shape), _full(WB.shape), _full((1, h2)),
                  _full(W2T.shape), _full((1, h_dim))],
        out_specs=pl.BlockSpec((blk, h_dim), lambda i: (i, 0)),
        out_shape=jax.ShapeDtypeStruct((ep, h_dim), jnp.float32),
    )(xi, xj, WA, WB, bc1[None], W2T, bc2[None])


def _edge_score(hs, hd, WA, WB, be1, W2T, be2, blk):
    ep = hs.shape[0]
    h_dim = WA.shape[1]
    return pl.pallas_call(
        _edge_score_kernel,
        grid=(ep // blk,),
        in_specs=[pl.BlockSpec((blk, hs.shape[1]), lambda i: (i, 0)),
                  pl.BlockSpec((blk, hd.shape[1]), lambda i: (i, 0)),
                  _full(WA.shape), _full(WB.shape), _full((1, h_dim)),
                  _full(W2T.shape), _full((1, 1))],
        out_specs=pl.BlockSpec((blk, 1), lambda i: (i, 0)),
        out_shape=jax.ShapeDtypeStruct((ep, 1), jnp.float32),
    )(hs, hd, WA, WB, be1[None], W2T, be2[None])


def _pad_to(a, size, fill):
    return jnp.concatenate([a, jnp.full((size - a.shape[0],), fill, a.dtype)])


def kernel(x, edge_index, W1, b1, bn_g, bn_b, bn_rm, bn_rv, W2, b2, Wc1, bc1, Wc2, bc2, We1, be1, We2, be2):
    n = x.shape[0]
    e_cnt = edge_index.shape[1]
    h_dim = W1.shape[0]
    nt = n + 16

    loops = jnp.arange(n, dtype=edge_index.dtype)
    ei = jnp.concatenate([edge_index, jnp.stack([loops, loops])], axis=1)
    ei2 = jnp.concatenate([ei, ei[::-1]], axis=1)
    k = ei2[0] * n + ei2[1]
    ks = jnp.sort(k)
    mask = jnp.concatenate([jnp.ones((1,), dtype=bool), ks[1:] != ks[:-1]])
    row = ks // n
    col = ks % n

    ec = ks.shape[0]
    grain = NW * ROWS_PER_TX * TX_PER_CHUNK
    ec_pad = ((ec + grain - 1) // grain) * grain
    rowp = _pad_to(row, ec_pad, 0).reshape(ec_pad // ROWS_PER_TX, ROWS_PER_TX)
    colp = _pad_to(col, ec_pad, 0).reshape(ec_pad // ROWS_PER_TX, ROWS_PER_TX)
    segp = _pad_to(jnp.where(mask, row, n), ec_pad, n).reshape(
        ec_pad // ROWS_PER_TX, ROWS_PER_TX)

    h = _input_net(x, W1, b1, bn_g, bn_b, bn_rm, bn_rv, W2, b2)

    A = Wc1[:, :h_dim]
    B = Wc1[:, h_dim:]
    WA = (A - B).T
    WB = B.T
    W2T = Wc2.T

    gather2 = _make_gather2(n, h_dim, ec_pad)
    scatter_add = _make_scatter_add(nt, h_dim, ec_pad)
    zeros_nt = jnp.zeros((nt, h_dim), jnp.float32)
    blk = 8192

    for _ in range(3):
        xi, xj = gather2(h, rowp, colp)
        m = _edge_mlp(xi, xj, WA, WB, bc1, W2T, bc2, blk)
        partials = scatter_add(m, segp, zeros_nt)
        h = partials[0, :n] + partials[1, :n]

    e_pad = ((e_cnt + grain - 1) // grain) * grain
    srcp = _pad_to(edge_index[0], e_pad, 0).reshape(e_pad // ROWS_PER_TX, ROWS_PER_TX)
    dstp = _pad_to(edge_index[1], e_pad, 0).reshape(e_pad // ROWS_PER_TX, ROWS_PER_TX)
    hs, hd = gather2(h, srcp, dstp) if e_pad == ec_pad else _make_gather2(n, h_dim, e_pad)(h, srcp, dstp)
    EA = We1[:, :h_dim].T
    EB = We1[:, h_dim:].T
    scores = _edge_score(hs, hd, EA, EB, be1, We2.T, be2, blk)
    return scores[:e_cnt, 0]


# speedup vs baseline: 4.0692x; 1.4809x over previous
import functools

import jax
import jax.numpy as jnp
from jax import lax
from jax.experimental import pallas as pl
from jax.experimental.pallas import tpu as pltpu
from jax.experimental.pallas import tpu_sc as plsc

NW = 32
NC = 2
ROWS_PER_TX = 128
TX_PER_CHUNK = 4


def _sc_mesh():
    return plsc.VectorSubcoreMesh(core_axis_name="c", subcore_axis_name="s")


def _make_gather2(n_pad, h_dim, ec_pad):
    chunk = ROWS_PER_TX * TX_PER_CHUNK
    per_w = ec_pad // NW
    n_chunks = per_w // chunk
    assert per_w % chunk == 0 and n_chunks % 2 == 0

    @functools.partial(
        pl.kernel,
        out_type=[jax.ShapeDtypeStruct((ec_pad, h_dim), jnp.float32),
                  jax.ShapeDtypeStruct((ec_pad, h_dim), jnp.float32)],
        mesh=_sc_mesh(),
        compiler_params=pltpu.CompilerParams(use_tc_tiling_on_sc=False),
        scratch_types=[
            [pltpu.VMEM((TX_PER_CHUNK, ROWS_PER_TX), jnp.int32) for _ in range(2)],
            [pltpu.VMEM((TX_PER_CHUNK, ROWS_PER_TX), jnp.int32) for _ in range(2)],
            [pltpu.VMEM((chunk, h_dim), jnp.float32) for _ in range(2)],
            [pltpu.VMEM((chunk, h_dim), jnp.float32) for _ in range(2)],
            [pltpu.SemaphoreType.DMA for _ in range(2)],
            [pltpu.SemaphoreType.DMA for _ in range(2)],
            [pltpu.SemaphoreType.DMA for _ in range(2)],
        ],
    )
    def gather2(tab_hbm, rows_hbm, cols_hbm, xi_hbm, xj_hbm, ia, ib, ba, bb,
                sidx, sg, sf):
        wid = lax.axis_index("s") * NC + lax.axis_index("c")
        base_r = wid * (per_w // ROWS_PER_TX)

        def idx_copies(ci, b):
            r0 = base_r + ci * TX_PER_CHUNK
            return (pltpu.make_async_copy(rows_hbm.at[pl.ds(r0, TX_PER_CHUNK)], ia[b], sidx[b]),
                    pltpu.make_async_copy(cols_hbm.at[pl.ds(r0, TX_PER_CHUNK)], ib[b], sidx[b]))

        def flush_copies(ci, b):
            off = (base_r + ci * TX_PER_CHUNK) * ROWS_PER_TX
            return (pltpu.make_async_copy(ba[b], xi_hbm.at[pl.ds(off, chunk)], sf[b]),
                    pltpu.make_async_copy(bb[b], xj_hbm.at[pl.ds(off, chunk)], sf[b]))

        for b in range(2):
            for cp in idx_copies(b, b):
                cp.start()

        @pl.loop(0, n_chunks, step=2)
        def _(i):
            for b in range(2):
                ci = i + b
                for cp in idx_copies(ci, b):
                    cp.wait()

                @pl.when(ci + 2 < n_chunks)
                def _():
                    for cp in idx_copies(ci + 2, b):
                        cp.start()

                @pl.when(ci >= 2)
                def _():
                    for cp in flush_copies(ci, b):
                        cp.wait()

                gathers = []
                for j in range(TX_PER_CHUNK):
                    gathers.append(pltpu.async_copy(
                        tab_hbm.at[ia[b].at[j]],
                        ba[b].at[pl.ds(j * ROWS_PER_TX, ROWS_PER_TX)], sg[b]))
                    gathers.append(pltpu.async_copy(
                        tab_hbm.at[ib[b].at[j]],
                        bb[b].at[pl.ds(j * ROWS_PER_TX, ROWS_PER_TX)], sg[b]))
                for cp in gathers:
                    cp.wait()
                for cp in flush_copies(ci, b):
                    cp.start()

        for b in range(2):
            for cp in flush_copies(n_chunks - 2 + b, b):
                cp.wait()

    return gather2


def _make_scatter_add(nt, h_dim, ec_pad):
    chunk = ROWS_PER_TX * TX_PER_CHUNK
    per_w = ec_pad // NW
    n_chunks = per_w // chunk
    rows_per_tile = nt // 16
    assert per_w % chunk == 0 and nt % 16 == 0

    @functools.partial(
        pl.kernel,
        out_type=jax.ShapeDtypeStruct((NC, nt, h_dim), jnp.float32),
        mesh=_sc_mesh(),
        compiler_params=pltpu.CompilerParams(use_tc_tiling_on_sc=False),
        scratch_types=[
            pltpu.VMEM((TX_PER_CHUNK, ROWS_PER_TX), jnp.int32),
            pltpu.VMEM((chunk, h_dim), jnp.float32),
            pltpu.VMEM_SHARED((nt, h_dim), jnp.float32),
        ],
    )
    def scatter_add(m_hbm, seg_hbm, zeros_hbm, out_hbm, idx, buf, acc):
        cid = lax.axis_index("c")
        sid = lax.axis_index("s")
        wid = sid * NC + cid
        base_r = wid * (per_w // ROWS_PER_TX)

        @pl.when(sid == 0)
        def _():
            pltpu.sync_copy(zeros_hbm, acc)

        plsc.subcore_barrier()

        @pl.loop(0, n_chunks)
        def _(i):
            r0 = base_r + i * TX_PER_CHUNK
            off = r0 * ROWS_PER_TX
            pltpu.sync_copy(seg_hbm.at[pl.ds(r0, TX_PER_CHUNK)], idx)
            pltpu.sync_copy(m_hbm.at[pl.ds(off, chunk)], buf)
            for j in range(TX_PER_CHUNK):
                pltpu.sync_copy(
                    buf.at[pl.ds(j * ROWS_PER_TX, ROWS_PER_TX)],
                    acc.at[idx.at[j]], add=True)

        plsc.subcore_barrier()
        pltpu.sync_copy(acc.at[pl.ds(sid * rows_per_tile, rows_per_tile)],
                        out_hbm.at[cid].at[pl.ds(sid * rows_per_tile, rows_per_tile)])

    return scatter_add


def _input_net_kernel(x_ref, w1_ref, scale_ref, shift_ref, w2_ref, b2_ref, o_ref):
    h = jnp.dot(x_ref[...], w1_ref[...], preferred_element_type=jnp.float32)
    h = h * scale_ref[...] + shift_ref[...]
    h = jnp.tanh(h)
    h = jnp.dot(h, w2_ref[...], preferred_element_type=jnp.float32) + b2_ref[...]
    o_ref[...] = jnp.maximum(h, 0.0)


def _edge_mlp_kernel(xi_ref, xj_ref, wa_ref, wb_ref, bc1_ref, w2_ref, bc2_ref, o_ref):
    m = jnp.dot(xi_ref[...], wa_ref[...], preferred_element_type=jnp.float32)
    m += jnp.dot(xj_ref[...], wb_ref[...], preferred_element_type=jnp.float32)
    m = jnp.maximum(m + bc1_ref[...], 0.0)
    m = jnp.dot(m, w2_ref[...], preferred_element_type=jnp.float32) + bc2_ref[...]
    o_ref[...] = jnp.maximum(m, 0.0)


def _edge_score_kernel(hs_ref, hd_ref, wa_ref, wb_ref, be1_ref, w2_ref, be2_ref, o_ref):
    e = jnp.dot(hs_ref[...], wa_ref[...], preferred_element_type=jnp.float32)
    e += jnp.dot(hd_ref[...], wb_ref[...], preferred_element_type=jnp.float32)
    e = jnp.maximum(e + be1_ref[...], 0.0)
    e = jnp.dot(e, w2_ref[...], preferred_element_type=jnp.float32) + be2_ref[...]
    o_ref[...] = jax.nn.sigmoid(e)


def _blockdiag(w, copies):
    a, b = w.shape
    eye = jnp.eye(copies, dtype=w.dtype)
    return (eye[:, None, :, None] * w[None, :, None, :]).reshape(copies * a, copies * b)


def _full(shape):
    return pl.BlockSpec(shape, lambda i: tuple(0 for _ in shape))


def _input_net(x, W1, b1, bn_g, bn_b, bn_rm, bn_rv, W2, b2):
    n, d = x.shape
    h_dim = W1.shape[0]
    inv = bn_g / jnp.sqrt(bn_rv + 1e-5)
    scale = inv
    shift = (b1 - bn_rm) * inv + bn_b
    return pl.pallas_call(
        _input_net_kernel,
        grid=(1,),
        in_specs=[_full((n, d)), _full((d, h_dim)), _full((1, h_dim)),
                  _full((1, h_dim)), _full((h_dim, h_dim)), _full((1, h_dim))],
        out_specs=_full((n, h_dim)),
        out_shape=jax.ShapeDtypeStruct((n, h_dim), jnp.float32),
    )(x, W1.T, scale[None], shift[None], W2.T, b2[None])


def _edge_mlp(xi, xj, WA, WB, bc1, W2T, bc2, blk):
    ep, h2 = xi.shape[0], WA.shape[1]
    h_dim = W2T.shape[1]
    return pl.pallas_call(
        _edge_mlp_kernel,
        grid=(ep // blk,),
        in_specs=[pl.BlockSpec((blk, xi.shape[1]), lambda i: (i, 0)),
                  pl.BlockSpec((blk, xj.shape[1]), lambda i: (i, 0)),
                  _full(WA.shape), _full(WB.shape), _full((1, h2)),
                  _full(W2T.shape), _full((1, h_dim))],
        out_specs=pl.BlockSpec((blk, h_dim), lambda i: (i, 0)),
        out_shape=jax.ShapeDtypeStruct((ep, h_dim), jnp.float32),
    )(xi, xj, WA, WB, bc1[None], W2T, bc2[None])


def _edge_score(hs, hd, WA, WB, be1, W2T, be2, blk):
    ep = hs.shape[0]
    h_dim = WA.shape[1]
    return pl.pallas_call(
        _edge_score_kernel,
        grid=(ep // blk,),
        in_specs=[pl.BlockSpec((blk, hs.shape[1]), lambda i: (i, 0)),
                  pl.BlockSpec((blk, hd.shape[1]), lambda i: (i, 0)),
                  _full(WA.shape), _full(WB.shape), _full((1, h_dim)),
                  _full(W2T.shape), _full((1, W2T.shape[1]))],
        out_specs=pl.BlockSpec((blk, W2T.shape[1]), lambda i: (i, 0)),
        out_shape=jax.ShapeDtypeStruct((ep, W2T.shape[1]), jnp.float32),
    )(hs, hd, WA, WB, be1[None], W2T, be2[None])


def _pad_to(a, size, fill):
    return jnp.concatenate([a, jnp.full((size - a.shape[0],), fill, a.dtype)])


def kernel(x, edge_index, W1, b1, bn_g, bn_b, bn_rm, bn_rv, W2, b2, Wc1, bc1, Wc2, bc2, We1, be1, We2, be2):
    n = x.shape[0]
    e_cnt = edge_index.shape[1]
    h_dim = W1.shape[0]
    nt = n + 16

    loops = jnp.arange(n, dtype=edge_index.dtype)
    ei = jnp.concatenate([edge_index, jnp.stack([loops, loops])], axis=1)
    ei2 = jnp.concatenate([ei, ei[::-1]], axis=1)
    k = ei2[0] * n + ei2[1]
    ks = jnp.sort(k)
    mask = jnp.concatenate([jnp.ones((1,), dtype=bool), ks[1:] != ks[:-1]])
    row = ks // n
    col = ks % n

    ec = ks.shape[0]
    grain = NW * ROWS_PER_TX * TX_PER_CHUNK * 2
    ec_pad = ((ec + grain - 1) // grain) * grain
    rowp = _pad_to(row, ec_pad, 0).reshape(ec_pad // ROWS_PER_TX, ROWS_PER_TX)
    colp = _pad_to(col, ec_pad, 0).reshape(ec_pad // ROWS_PER_TX, ROWS_PER_TX)
    segp = _pad_to(jnp.where(mask, row, n), ec_pad, n).reshape(
        ec_pad // ROWS_PER_TX, ROWS_PER_TX)

    h = _input_net(x, W1, b1, bn_g, bn_b, bn_rm, bn_rv, W2, b2)

    A = Wc1[:, :h_dim]
    B = Wc1[:, h_dim:]
    WA4 = _blockdiag((A - B).T, 4)
    WB4 = _blockdiag(B.T, 4)
    W2T4 = _blockdiag(Wc2.T, 4)
    bc1t = jnp.tile(bc1, 4)
    bc2t = jnp.tile(bc2, 4)

    gather2 = _make_gather2(n, h_dim, ec_pad)
    scatter_add = _make_scatter_add(nt, h_dim, ec_pad)
    zeros_nt = jnp.zeros((nt, h_dim), jnp.float32)
    blk4 = 2048

    for _ in range(3):
        xi, xj = gather2(h, rowp, colp)
        xi4 = xi.reshape(ec_pad // 4, 4 * h_dim)
        xj4 = xj.reshape(ec_pad // 4, 4 * h_dim)
        m4 = _edge_mlp(xi4, xj4, WA4, WB4, bc1t, W2T4, bc2t, blk4)
        m = m4.reshape(ec_pad, h_dim)
        partials = scatter_add(m, segp, zeros_nt)
        h = partials[0, :n] + partials[1, :n]

    e_pad = ((e_cnt + grain - 1) // grain) * grain
    srcp = _pad_to(edge_index[0], e_pad, 0).reshape(e_pad // ROWS_PER_TX, ROWS_PER_TX)
    dstp = _pad_to(edge_index[1], e_pad, 0).reshape(e_pad // ROWS_PER_TX, ROWS_PER_TX)
    hs, hd = gather2(h, srcp, dstp) if e_pad == ec_pad else _make_gather2(n, h_dim, e_pad)(h, srcp, dstp)
    EA4 = _blockdiag(We1[:, :h_dim].T, 4)
    EB4 = _blockdiag(We1[:, h_dim:].T, 4)
    W2c4 = _blockdiag(We2.T, 4)
    be1t = jnp.tile(be1, 4)
    be2t = jnp.tile(be2, 4)
    hs4 = hs.reshape(e_pad // 4, 4 * h_dim)
    hd4 = hd.reshape(e_pad // 4, 4 * h_dim)
    scores4 = _edge_score(hs4, hd4, EA4, EB4, be1t, W2c4, be2t, blk4)
    return scores4.reshape(e_pad)[:e_cnt]


# speedup vs baseline: 4.1055x; 1.0089x over previous
import functools

import jax
import jax.numpy as jnp
from jax import lax
from jax.experimental import pallas as pl
from jax.experimental.pallas import tpu as pltpu
from jax.experimental.pallas import tpu_sc as plsc

NW = 32
NC = 2
ROWS_PER_TX = 128
TX_PER_CHUNK = 4


def _sc_mesh():
    return plsc.VectorSubcoreMesh(core_axis_name="c", subcore_axis_name="s")


def _make_gather2(n_pad, h_dim, ec_pad):
    chunk = ROWS_PER_TX * TX_PER_CHUNK
    per_w = ec_pad // NW
    n_chunks = per_w // chunk
    assert per_w % chunk == 0 and n_chunks % 2 == 0

    @functools.partial(
        pl.kernel,
        out_type=[jax.ShapeDtypeStruct((ec_pad, h_dim), jnp.float32),
                  jax.ShapeDtypeStruct((ec_pad, h_dim), jnp.float32)],
        mesh=_sc_mesh(),
        compiler_params=pltpu.CompilerParams(use_tc_tiling_on_sc=False),
        scratch_types=[
            [pltpu.VMEM((TX_PER_CHUNK, ROWS_PER_TX), jnp.int32) for _ in range(2)],
            [pltpu.VMEM((TX_PER_CHUNK, ROWS_PER_TX), jnp.int32) for _ in range(2)],
            [pltpu.VMEM((chunk, h_dim), jnp.float32) for _ in range(2)],
            [pltpu.VMEM((chunk, h_dim), jnp.float32) for _ in range(2)],
            [pltpu.SemaphoreType.DMA for _ in range(2)],
            [pltpu.SemaphoreType.DMA for _ in range(2)],
            [pltpu.SemaphoreType.DMA for _ in range(2)],
        ],
    )
    def gather2(tab_hbm, rows_hbm, cols_hbm, xi_hbm, xj_hbm, ia, ib, ba, bb,
                sidx, sg, sf):
        wid = lax.axis_index("s") * NC + lax.axis_index("c")
        base_r = wid * (per_w // ROWS_PER_TX)

        def idx_copies(ci, b):
            r0 = base_r + ci * TX_PER_CHUNK
            return (pltpu.make_async_copy(rows_hbm.at[pl.ds(r0, TX_PER_CHUNK)], ia[b], sidx[b]),
                    pltpu.make_async_copy(cols_hbm.at[pl.ds(r0, TX_PER_CHUNK)], ib[b], sidx[b]))

        def flush_copies(ci, b):
            off = (base_r + ci * TX_PER_CHUNK) * ROWS_PER_TX
            return (pltpu.make_async_copy(ba[b], xi_hbm.at[pl.ds(off, chunk)], sf[b]),
                    pltpu.make_async_copy(bb[b], xj_hbm.at[pl.ds(off, chunk)], sf[b]))

        for b in range(2):
            for cp in idx_copies(b, b):
                cp.start()

        @pl.loop(0, n_chunks, step=2)
        def _(i):
            for b in range(2):
                ci = i + b
                for cp in idx_copies(ci, b):
                    cp.wait()

                @pl.when(ci >= 2)
                def _():
                    for cp in flush_copies(ci, b):
                        cp.wait()

                gathers = []
                for j in range(TX_PER_CHUNK):
                    gathers.append(pltpu.async_copy(
                        tab_hbm.at[ia[b].at[j]],
                        ba[b].at[pl.ds(j * ROWS_PER_TX, ROWS_PER_TX)], sg[b]))
                    gathers.append(pltpu.async_copy(
                        tab_hbm.at[ib[b].at[j]],
                        bb[b].at[pl.ds(j * ROWS_PER_TX, ROWS_PER_TX)], sg[b]))
                for cp in gathers:
                    cp.wait()

                @pl.when(ci + 2 < n_chunks)
                def _():
                    for cp in idx_copies(ci + 2, b):
                        cp.start()

                for cp in flush_copies(ci, b):
                    cp.start()

        for b in range(2):
            for cp in flush_copies(n_chunks - 2 + b, b):
                cp.wait()

    return gather2


def _make_scatter_add(nt, h_dim, ec_pad):
    chunk = ROWS_PER_TX * TX_PER_CHUNK
    per_w = ec_pad // NW
    n_chunks = per_w // chunk
    rows_per_tile = nt // 16
    assert per_w % chunk == 0 and nt % 16 == 0

    @functools.partial(
        pl.kernel,
        out_type=jax.ShapeDtypeStruct((NC, nt, h_dim), jnp.float32),
        mesh=_sc_mesh(),
        compiler_params=pltpu.CompilerParams(use_tc_tiling_on_sc=False),
        scratch_types=[
            pltpu.VMEM((TX_PER_CHUNK, ROWS_PER_TX), jnp.int32),
            pltpu.VMEM((chunk, h_dim), jnp.float32),
            pltpu.VMEM_SHARED((nt, h_dim), jnp.float32),
        ],
    )
    def scatter_add(m_hbm, seg_hbm, zeros_hbm, out_hbm, idx, buf, acc):
        cid = lax.axis_index("c")
        sid = lax.axis_index("s")
        wid = sid * NC + cid
        base_r = wid * (per_w // ROWS_PER_TX)

        @pl.when(sid == 0)
        def _():
            pltpu.sync_copy(zeros_hbm, acc)

        plsc.subcore_barrier()

        @pl.loop(0, n_chunks)
        def _(i):
            r0 = base_r + i * TX_PER_CHUNK
            off = r0 * ROWS_PER_TX
            pltpu.sync_copy(seg_hbm.at[pl.ds(r0, TX_PER_CHUNK)], idx)
            pltpu.sync_copy(m_hbm.at[pl.ds(off, chunk)], buf)
            for j in range(TX_PER_CHUNK):
                pltpu.sync_copy(
                    buf.at[pl.ds(j * ROWS_PER_TX, ROWS_PER_TX)],
                    acc.at[idx.at[j]], add=True)

        plsc.subcore_barrier()
        pltpu.sync_copy(acc.at[pl.ds(sid * rows_per_tile, rows_per_tile)],
                        out_hbm.at[cid].at[pl.ds(sid * rows_per_tile, rows_per_tile)])

    return scatter_add


def _input_net_kernel(x_ref, w1_ref, scale_ref, shift_ref, w2_ref, b2_ref, o_ref):
    h = jnp.dot(x_ref[...], w1_ref[...], preferred_element_type=jnp.float32)
    h = h * scale_ref[...] + shift_ref[...]
    h = jnp.tanh(h)
    h = jnp.dot(h, w2_ref[...], preferred_element_type=jnp.float32) + b2_ref[...]
    o_ref[...] = jnp.maximum(h, 0.0)


def _edge_mlp_kernel(xi_ref, xj_ref, wa_ref, wb_ref, bc1_ref, w2_ref, bc2_ref, o_ref):
    m = jnp.dot(xi_ref[...], wa_ref[...], preferred_element_type=jnp.float32)
    m += jnp.dot(xj_ref[...], wb_ref[...], preferred_element_type=jnp.float32)
    m = jnp.maximum(m + bc1_ref[...], 0.0)
    m = jnp.dot(m, w2_ref[...], preferred_element_type=jnp.float32) + bc2_ref[...]
    o_ref[...] = jnp.maximum(m, 0.0)


def _edge_score_kernel(hs_ref, hd_ref, wa_ref, wb_ref, be1_ref, w2_ref, be2_ref, o_ref):
    e = jnp.dot(hs_ref[...], wa_ref[...], preferred_element_type=jnp.float32)
    e += jnp.dot(hd_ref[...], wb_ref[...], preferred_element_type=jnp.float32)
    e = jnp.maximum(e + be1_ref[...], 0.0)
    e = jnp.dot(e, w2_ref[...], preferred_element_type=jnp.float32) + be2_ref[...]
    o_ref[...] = jax.nn.sigmoid(e)


def _blockdiag(w, copies):
    a, b = w.shape
    eye = jnp.eye(copies, dtype=w.dtype)
    return (eye[:, None, :, None] * w[None, :, None, :]).reshape(copies * a, copies * b)


def _full(shape):
    return pl.BlockSpec(shape, lambda i: tuple(0 for _ in shape))


def _input_net(x, W1, b1, bn_g, bn_b, bn_rm, bn_rv, W2, b2):
    n, d = x.shape
    h_dim = W1.shape[0]
    inv = bn_g / jnp.sqrt(bn_rv + 1e-5)
    scale = inv
    shift = (b1 - bn_rm) * inv + bn_b
    return pl.pallas_call(
        _input_net_kernel,
        grid=(1,),
        in_specs=[_full((n, d)), _full((d, h_dim)), _full((1, h_dim)),
                  _full((1, h_dim)), _full((h_dim, h_dim)), _full((1, h_dim))],
        out_specs=_full((n, h_dim)),
        out_shape=jax.ShapeDtypeStruct((n, h_dim), jnp.float32),
    )(x, W1.T, scale[None], shift[None], W2.T, b2[None])


def _edge_mlp(xi, xj, WA, WB, bc1, W2T, bc2, blk):
    ep, h2 = xi.shape[0], WA.shape[1]
    h_dim = W2T.shape[1]
    return pl.pallas_call(
        _edge_mlp_kernel,
        grid=(ep // blk,),
        in_specs=[pl.BlockSpec((blk, xi.shape[1]), lambda i: (i, 0)),
                  pl.BlockSpec((blk, xj.shape[1]), lambda i: (i, 0)),
                  _full(WA.shape), _full(WB.shape), _full((1, h2)),
                  _full(W2T.shape), _full((1, h_dim))],
        out_specs=pl.BlockSpec((blk, h_dim), lambda i: (i, 0)),
        out_shape=jax.ShapeDtypeStruct((ep, h_dim), jnp.float32),
    )(xi, xj, WA, WB, bc1[None], W2T, bc2[None])


def _edge_score(hs, hd, WA, WB, be1, W2T, be2, blk):
    ep = hs.shape[0]
    h_dim = WA.shape[1]
    return pl.pallas_call(
        _edge_score_kernel,
        grid=(ep // blk,),
        in_specs=[pl.BlockSpec((blk, hs.shape[1]), lambda i: (i, 0)),
                  pl.BlockSpec((blk, hd.shape[1]), lambda i: (i, 0)),
                  _full(WA.shape), _full(WB.shape), _full((1, h_dim)),
                  _full(W2T.shape), _full((1, W2T.shape[1]))],
        out_specs=pl.BlockSpec((blk, W2T.shape[1]), lambda i: (i, 0)),
        out_shape=jax.ShapeDtypeStruct((ep, W2T.shape[1]), jnp.float32),
    )(hs, hd, WA, WB, be1[None], W2T, be2[None])


def _pad_to(a, size, fill):
    return jnp.concatenate([a, jnp.full((size - a.shape[0],), fill, a.dtype)])


def kernel(x, edge_index, W1, b1, bn_g, bn_b, bn_rm, bn_rv, W2, b2, Wc1, bc1, Wc2, bc2, We1, be1, We2, be2):
    n = x.shape[0]
    e_cnt = edge_index.shape[1]
    h_dim = W1.shape[0]
    nt = n + 16

    loops = jnp.arange(n, dtype=edge_index.dtype)
    ei = jnp.concatenate([edge_index, jnp.stack([loops, loops])], axis=1)
    ei2 = jnp.concatenate([ei, ei[::-1]], axis=1)
    k = ei2[0] * n + ei2[1]
    ks = jnp.sort(k)
    mask = jnp.concatenate([jnp.ones((1,), dtype=bool), ks[1:] != ks[:-1]])
    row = ks // n
    col = ks % n

    ec = ks.shape[0]
    grain = NW * ROWS_PER_TX * TX_PER_CHUNK * 2
    ec_pad = ((ec + grain - 1) // grain) * grain
    rowp = _pad_to(row, ec_pad, 0).reshape(ec_pad // ROWS_PER_TX, ROWS_PER_TX)
    colp = _pad_to(col, ec_pad, 0).reshape(ec_pad // ROWS_PER_TX, ROWS_PER_TX)
    segp = _pad_to(jnp.where(mask, row, n), ec_pad, n).reshape(
        ec_pad // ROWS_PER_TX, ROWS_PER_TX)

    h = _input_net(x, W1, b1, bn_g, bn_b, bn_rm, bn_rv, W2, b2)

    A = Wc1[:, :h_dim]
    B = Wc1[:, h_dim:]
    WA4 = _blockdiag((A - B).T, 4)
    WB4 = _blockdiag(B.T, 4)
    W2T4 = _blockdiag(Wc2.T, 4)
    bc1t = jnp.tile(bc1, 4)
    bc2t = jnp.tile(bc2, 4)

    gather2 = _make_gather2(n, h_dim, ec_pad)
    scatter_add = _make_scatter_add(nt, h_dim, ec_pad)
    zeros_nt = jnp.zeros((nt, h_dim), jnp.float32)
    blk4 = 2048

    for _ in range(3):
        xi, xj = gather2(h, rowp, colp)
        xi4 = xi.reshape(ec_pad // 4, 4 * h_dim)
        xj4 = xj.reshape(ec_pad // 4, 4 * h_dim)
        m4 = _edge_mlp(xi4, xj4, WA4, WB4, bc1t, W2T4, bc2t, blk4)
        m = m4.reshape(ec_pad, h_dim)
        partials = scatter_add(m, segp, zeros_nt)
        h = partials[0, :n] + partials[1, :n]

    e_pad = ((e_cnt + grain - 1) // grain) * grain
    srcp = _pad_to(edge_index[0], e_pad, 0).reshape(e_pad // ROWS_PER_TX, ROWS_PER_TX)
    dstp = _pad_to(edge_index[1], e_pad, 0).reshape(e_pad // ROWS_PER_TX, ROWS_PER_TX)
    hs, hd = gather2(h, srcp, dstp) if e_pad == ec_pad else _make_gather2(n, h_dim, e_pad)(h, srcp, dstp)
    EA4 = _blockdiag(We1[:, :h_dim].T, 4)
    EB4 = _blockdiag(We1[:, h_dim:].T, 4)
    W2c4 = _blockdiag(We2.T, 4)
    be1t = jnp.tile(be1, 4)
    be2t = jnp.tile(be2, 4)
    hs4 = hs.reshape(e_pad // 4, 4 * h_dim)
    hd4 = hd.reshape(e_pad // 4, 4 * h_dim)
    scores4 = _edge_score(hs4, hd4, EA4, EB4, be1t, W2c4, be2t, blk4)
    return scores4.reshape(e_pad)[:e_cnt]


# speedup vs baseline: 4.1269x; 1.0052x over previous
import functools

import jax
import jax.numpy as jnp
from jax import lax
from jax.experimental import pallas as pl
from jax.experimental.pallas import tpu as pltpu
from jax.experimental.pallas import tpu_sc as plsc

NW = 32
NC = 2
ROWS_PER_TX = 128
TX_PER_CHUNK = 4


def _sc_mesh():
    return plsc.VectorSubcoreMesh(core_axis_name="c", subcore_axis_name="s")


def _make_gather2(n_pad, h_dim, ec_pad):
    chunk = ROWS_PER_TX * TX_PER_CHUNK
    per_w = ec_pad // NW
    n_chunks = per_w // chunk
    assert per_w % chunk == 0 and n_chunks % 2 == 0

    @functools.partial(
        pl.kernel,
        out_type=[jax.ShapeDtypeStruct((ec_pad, h_dim), jnp.float32),
                  jax.ShapeDtypeStruct((ec_pad, h_dim), jnp.float32)],
        mesh=_sc_mesh(),
        compiler_params=pltpu.CompilerParams(use_tc_tiling_on_sc=False),
        scratch_types=[
            [pltpu.VMEM((TX_PER_CHUNK, ROWS_PER_TX), jnp.int32) for _ in range(2)],
            [pltpu.VMEM((TX_PER_CHUNK, ROWS_PER_TX), jnp.int32) for _ in range(2)],
            [pltpu.VMEM((chunk, h_dim), jnp.float32) for _ in range(2)],
            [pltpu.VMEM((chunk, h_dim), jnp.float32) for _ in range(2)],
            [pltpu.SemaphoreType.DMA for _ in range(2)],
            [pltpu.SemaphoreType.DMA for _ in range(2)],
            [pltpu.SemaphoreType.DMA for _ in range(2)],
        ],
    )
    def gather2(tab_hbm, rows_hbm, cols_hbm, xi_hbm, xj_hbm, ia, ib, ba, bb,
                sidx, sg, sf):
        wid = lax.axis_index("s") * NC + lax.axis_index("c")
        base_r = wid * (per_w // ROWS_PER_TX)

        def idx_copies(ci, b):
            r0 = base_r + ci * TX_PER_CHUNK
            return (pltpu.make_async_copy(rows_hbm.at[pl.ds(r0, TX_PER_CHUNK)], ia[b], sidx[b]),
                    pltpu.make_async_copy(cols_hbm.at[pl.ds(r0, TX_PER_CHUNK)], ib[b], sidx[b]))

        def gather_copies(b):
            cps = []
            for j in range(TX_PER_CHUNK):
                cps.append(pltpu.make_async_copy(
                    tab_hbm.at[ia[b].at[j]],
                    ba[b].at[pl.ds(j * ROWS_PER_TX, ROWS_PER_TX)], sg[b]))
                cps.append(pltpu.make_async_copy(
                    tab_hbm.at[ib[b].at[j]],
                    bb[b].at[pl.ds(j * ROWS_PER_TX, ROWS_PER_TX)], sg[b]))
            return cps

        def flush_copies(ci, b):
            off = (base_r + ci * TX_PER_CHUNK) * ROWS_PER_TX
            return (pltpu.make_async_copy(ba[b], xi_hbm.at[pl.ds(off, chunk)], sf[b]),
                    pltpu.make_async_copy(bb[b], xj_hbm.at[pl.ds(off, chunk)], sf[b]))

        for cp in idx_copies(0, 0):
            cp.start()

        @pl.loop(0, n_chunks, step=2)
        def _(i):
            for b in range(2):
                ci = i + b
                for cp in idx_copies(ci, b):
                    cp.wait()

                @pl.when(ci >= 2)
                def _():
                    for cp in flush_copies(ci, b):
                        cp.wait()

                for cp in gather_copies(b):
                    cp.start()

                @pl.when(ci >= 1)
                def _():
                    for cp in gather_copies(1 - b):
                        cp.wait()
                    for cp in flush_copies(ci - 1, 1 - b):
                        cp.start()

                @pl.when(ci + 1 < n_chunks)
                def _():
                    for cp in idx_copies(ci + 1, 1 - b):
                        cp.start()

        last = n_chunks - 1
        for cp in gather_copies(last % 2):
            cp.wait()
        for cp in flush_copies(last, last % 2):
            cp.start()
        for b in range(2):
            for cp in flush_copies(n_chunks - 2 + b, b):
                cp.wait()

    return gather2


def _make_scatter_add(nt, h_dim, ec_pad):
    chunk = ROWS_PER_TX * TX_PER_CHUNK
    per_w = ec_pad // NW
    n_chunks = per_w // chunk
    rows_per_tile = nt // 16
    assert per_w % chunk == 0 and nt % 16 == 0

    @functools.partial(
        pl.kernel,
        out_type=jax.ShapeDtypeStruct((NC, nt, h_dim), jnp.float32),
        mesh=_sc_mesh(),
        compiler_params=pltpu.CompilerParams(use_tc_tiling_on_sc=False),
        scratch_types=[
            pltpu.VMEM((TX_PER_CHUNK, ROWS_PER_TX), jnp.int32),
            pltpu.VMEM((chunk, h_dim), jnp.float32),
            pltpu.VMEM_SHARED((nt, h_dim), jnp.float32),
        ],
    )
    def scatter_add(m_hbm, seg_hbm, zeros_hbm, out_hbm, idx, buf, acc):
        cid = lax.axis_index("c")
        sid = lax.axis_index("s")
        wid = sid * NC + cid
        base_r = wid * (per_w // ROWS_PER_TX)

        @pl.when(sid == 0)
        def _():
            pltpu.sync_copy(zeros_hbm, acc)

        plsc.subcore_barrier()

        @pl.loop(0, n_chunks)
        def _(i):
            r0 = base_r + i * TX_PER_CHUNK
            off = r0 * ROWS_PER_TX
            pltpu.sync_copy(seg_hbm.at[pl.ds(r0, TX_PER_CHUNK)], idx)
            pltpu.sync_copy(m_hbm.at[pl.ds(off, chunk)], buf)
            for j in range(TX_PER_CHUNK):
                pltpu.sync_copy(
                    buf.at[pl.ds(j * ROWS_PER_TX, ROWS_PER_TX)],
                    acc.at[idx.at[j]], add=True)

        plsc.subcore_barrier()
        pltpu.sync_copy(acc.at[pl.ds(sid * rows_per_tile, rows_per_tile)],
                        out_hbm.at[cid].at[pl.ds(sid * rows_per_tile, rows_per_tile)])

    return scatter_add


def _input_net_kernel(x_ref, w1_ref, scale_ref, shift_ref, w2_ref, b2_ref, o_ref):
    h = jnp.dot(x_ref[...], w1_ref[...], preferred_element_type=jnp.float32)
    h = h * scale_ref[...] + shift_ref[...]
    h = jnp.tanh(h)
    h = jnp.dot(h, w2_ref[...], preferred_element_type=jnp.float32) + b2_ref[...]
    o_ref[...] = jnp.maximum(h, 0.0)


def _edge_mlp_kernel(xi_ref, xj_ref, wa_ref, wb_ref, bc1_ref, w2_ref, bc2_ref, o_ref):
    m = jnp.dot(xi_ref[...], wa_ref[...], preferred_element_type=jnp.float32)
    m += jnp.dot(xj_ref[...], wb_ref[...], preferred_element_type=jnp.float32)
    m = jnp.maximum(m + bc1_ref[...], 0.0)
    m = jnp.dot(m, w2_ref[...], preferred_element_type=jnp.float32) + bc2_ref[...]
    o_ref[...] = jnp.maximum(m, 0.0)


def _edge_score_kernel(hs_ref, hd_ref, wa_ref, wb_ref, be1_ref, w2_ref, be2_ref, o_ref):
    e = jnp.dot(hs_ref[...], wa_ref[...], preferred_element_type=jnp.float32)
    e += jnp.dot(hd_ref[...], wb_ref[...], preferred_element_type=jnp.float32)
    e = jnp.maximum(e + be1_ref[...], 0.0)
    e = jnp.dot(e, w2_ref[...], preferred_element_type=jnp.float32) + be2_ref[...]
    o_ref[...] = jax.nn.sigmoid(e)


def _blockdiag(w, copies):
    a, b = w.shape
    eye = jnp.eye(copies, dtype=w.dtype)
    return (eye[:, None, :, None] * w[None, :, None, :]).reshape(copies * a, copies * b)


def _full(shape):
    return pl.BlockSpec(shape, lambda i: tuple(0 for _ in shape))


def _input_net(x, W1, b1, bn_g, bn_b, bn_rm, bn_rv, W2, b2):
    n, d = x.shape
    h_dim = W1.shape[0]
    inv = bn_g / jnp.sqrt(bn_rv + 1e-5)
    scale = inv
    shift = (b1 - bn_rm) * inv + bn_b
    return pl.pallas_call(
        _input_net_kernel,
        grid=(1,),
        in_specs=[_full((n, d)), _full((d, h_dim)), _full((1, h_dim)),
                  _full((1, h_dim)), _full((h_dim, h_dim)), _full((1, h_dim))],
        out_specs=_full((n, h_dim)),
        out_shape=jax.ShapeDtypeStruct((n, h_dim), jnp.float32),
    )(x, W1.T, scale[None], shift[None], W2.T, b2[None])


def _edge_mlp(xi, xj, WA, WB, bc1, W2T, bc2, blk):
    ep, h2 = xi.shape[0], WA.shape[1]
    h_dim = W2T.shape[1]
    return pl.pallas_call(
        _edge_mlp_kernel,
        grid=(ep // blk,),
        in_specs=[pl.BlockSpec((blk, xi.shape[1]), lambda i: (i, 0)),
                  pl.BlockSpec((blk, xj.shape[1]), lambda i: (i, 0)),
                  _full(WA.shape), _full(WB.shape), _full((1, h2)),
                  _full(W2T.shape), _full((1, h_dim))],
        out_specs=pl.BlockSpec((blk, h_dim), lambda i: (i, 0)),
        out_shape=jax.ShapeDtypeStruct((ep, h_dim), jnp.float32),
    )(xi, xj, WA, WB, bc1[None], W2T, bc2[None])


def _edge_score(hs, hd, WA, WB, be1, W2T, be2, blk):
    ep = hs.shape[0]
    h_dim = WA.shape[1]
    return pl.pallas_call(
        _edge_score_kernel,
        grid=(ep // blk,),
        in_specs=[pl.BlockSpec((blk, hs.shape[1]), lambda i: (i, 0)),
                  pl.BlockSpec((blk, hd.shape[1]), lambda i: (i, 0)),
                  _full(WA.shape), _full(WB.shape), _full((1, h_dim)),
                  _full(W2T.shape), _full((1, W2T.shape[1]))],
        out_specs=pl.BlockSpec((blk, W2T.shape[1]), lambda i: (i, 0)),
        out_shape=jax.ShapeDtypeStruct((ep, W2T.shape[1]), jnp.float32),
    )(hs, hd, WA, WB, be1[None], W2T, be2[None])


def _pad_to(a, size, fill):
    return jnp.concatenate([a, jnp.full((size - a.shape[0],), fill, a.dtype)])


def kernel(x, edge_index, W1, b1, bn_g, bn_b, bn_rm, bn_rv, W2, b2, Wc1, bc1, Wc2, bc2, We1, be1, We2, be2):
    n = x.shape[0]
    e_cnt = edge_index.shape[1]
    h_dim = W1.shape[0]
    nt = n + 16

    loops = jnp.arange(n, dtype=edge_index.dtype)
    ei = jnp.concatenate([edge_index, jnp.stack([loops, loops])], axis=1)
    ei2 = jnp.concatenate([ei, ei[::-1]], axis=1)
    k = ei2[0] * n + ei2[1]
    ks = jnp.sort(k)
    mask = jnp.concatenate([jnp.ones((1,), dtype=bool), ks[1:] != ks[:-1]])
    row = ks // n
    col = ks % n

    ec = ks.shape[0]
    grain = NW * ROWS_PER_TX * TX_PER_CHUNK * 2
    ec_pad = ((ec + grain - 1) // grain) * grain
    rowp = _pad_to(row, ec_pad, 0).reshape(ec_pad // ROWS_PER_TX, ROWS_PER_TX)
    colp = _pad_to(col, ec_pad, 0).reshape(ec_pad // ROWS_PER_TX, ROWS_PER_TX)
    segp = _pad_to(jnp.where(mask, row, n), ec_pad, n).reshape(
        ec_pad // ROWS_PER_TX, ROWS_PER_TX)

    h = _input_net(x, W1, b1, bn_g, bn_b, bn_rm, bn_rv, W2, b2)

    A = Wc1[:, :h_dim]
    B = Wc1[:, h_dim:]
    WA4 = _blockdiag((A - B).T, 4)
    WB4 = _blockdiag(B.T, 4)
    W2T4 = _blockdiag(Wc2.T, 4)
    bc1t = jnp.tile(bc1, 4)
    bc2t = jnp.tile(bc2, 4)

    gather2 = _make_gather2(n, h_dim, ec_pad)
    scatter_add = _make_scatter_add(nt, h_dim, ec_pad)
    zeros_nt = jnp.zeros((nt, h_dim), jnp.float32)
    blk4 = 2048

    for _ in range(3):
        xi, xj = gather2(h, rowp, colp)
        xi4 = xi.reshape(ec_pad // 4, 4 * h_dim)
        xj4 = xj.reshape(ec_pad // 4, 4 * h_dim)
        m4 = _edge_mlp(xi4, xj4, WA4, WB4, bc1t, W2T4, bc2t, blk4)
        m = m4.reshape(ec_pad, h_dim)
        partials = scatter_add(m, segp, zeros_nt)
        h = partials[0, :n] + partials[1, :n]

    e_pad = ((e_cnt + grain - 1) // grain) * grain
    srcp = _pad_to(edge_index[0], e_pad, 0).reshape(e_pad // ROWS_PER_TX, ROWS_PER_TX)
    dstp = _pad_to(edge_index[1], e_pad, 0).reshape(e_pad // ROWS_PER_TX, ROWS_PER_TX)
    hs, hd = gather2(h, srcp, dstp) if e_pad == ec_pad else _make_gather2(n, h_dim, e_pad)(h, srcp, dstp)
    EA4 = _blockdiag(We1[:, :h_dim].T, 4)
    EB4 = _blockdiag(We1[:, h_dim:].T, 4)
    W2c4 = _blockdiag(We2.T, 4)
    be1t = jnp.tile(be1, 4)
    be2t = jnp.tile(be2, 4)
    hs4 = hs.reshape(e_pad // 4, 4 * h_dim)
    hd4 = hd.reshape(e_pad // 4, 4 * h_dim)
    scores4 = _edge_score(hs4, hd4, EA4, EB4, be1t, W2c4, be2t, blk4)
    return scores4.reshape(e_pad)[:e_cnt]


# speedup vs baseline: 4.1547x; 1.0067x over previous
import functools

import jax
import jax.numpy as jnp
from jax import lax
from jax.experimental import pallas as pl
from jax.experimental.pallas import tpu as pltpu
from jax.experimental.pallas import tpu_sc as plsc

NW = 32
NC = 2
ROWS_PER_TX = 128
TX_PER_CHUNK = 4


def _sc_mesh():
    return plsc.VectorSubcoreMesh(core_axis_name="c", subcore_axis_name="s")


def _make_gather2(n_pad, h_dim, ec_pad):
    chunk = ROWS_PER_TX * TX_PER_CHUNK
    per_w = ec_pad // NW
    n_chunks = per_w // chunk
    assert per_w % chunk == 0 and n_chunks % 2 == 0

    @functools.partial(
        pl.kernel,
        out_type=[jax.ShapeDtypeStruct((ec_pad, h_dim), jnp.float32),
                  jax.ShapeDtypeStruct((ec_pad, h_dim), jnp.float32)],
        mesh=_sc_mesh(),
        compiler_params=pltpu.CompilerParams(use_tc_tiling_on_sc=False),
        scratch_types=[
            [pltpu.VMEM((chunk,), jnp.int32) for _ in range(2)],
            [pltpu.VMEM((chunk,), jnp.int32) for _ in range(2)],
            [pltpu.VMEM((chunk, h_dim), jnp.float32) for _ in range(2)],
            [pltpu.VMEM((chunk, h_dim), jnp.float32) for _ in range(2)],
            [pltpu.SemaphoreType.DMA for _ in range(2)],
            [pltpu.SemaphoreType.DMA for _ in range(2)],
            [pltpu.SemaphoreType.DMA for _ in range(2)],
        ],
    )
    def gather2(tab_hbm, rows_hbm, cols_hbm, xi_hbm, xj_hbm, ia, ib, ba, bb,
                sidx, sg, sf):
        wid = lax.axis_index("s") * NC + lax.axis_index("c")
        base = wid * per_w

        def idx_copies(ci, b):
            off = base + ci * chunk
            return (pltpu.make_async_copy(rows_hbm.at[pl.ds(off, chunk)], ia[b], sidx[b]),
                    pltpu.make_async_copy(cols_hbm.at[pl.ds(off, chunk)], ib[b], sidx[b]))

        def gather_copies(b):
            return (pltpu.make_async_copy(tab_hbm.at[ia[b]], ba[b], sg[b]),
                    pltpu.make_async_copy(tab_hbm.at[ib[b]], bb[b], sg[b]))

        def flush_copies(ci, b):
            off = base + ci * chunk
            return (pltpu.make_async_copy(ba[b], xi_hbm.at[pl.ds(off, chunk)], sf[b]),
                    pltpu.make_async_copy(bb[b], xj_hbm.at[pl.ds(off, chunk)], sf[b]))

        for cp in idx_copies(0, 0):
            cp.start()

        @pl.loop(0, n_chunks, step=2)
        def _(i):
            for b in range(2):
                ci = i + b
                for cp in idx_copies(ci, b):
                    cp.wait()

                @pl.when(ci >= 2)
                def _():
                    for cp in flush_copies(ci, b):
                        cp.wait()

                for cp in gather_copies(b):
                    cp.start()

                @pl.when(ci >= 1)
                def _():
                    for cp in gather_copies(1 - b):
                        cp.wait()
                    for cp in flush_copies(ci - 1, 1 - b):
                        cp.start()

                @pl.when(ci + 1 < n_chunks)
                def _():
                    for cp in idx_copies(ci + 1, 1 - b):
                        cp.start()

        last = n_chunks - 1
        for cp in gather_copies(last % 2):
            cp.wait()
        for cp in flush_copies(last, last % 2):
            cp.start()
        for b in range(2):
            for cp in flush_copies(n_chunks - 2 + b, b):
                cp.wait()

    return gather2


def _make_scatter_add(nt, h_dim, ec_pad):
    chunk = ROWS_PER_TX * TX_PER_CHUNK
    per_w = ec_pad // NW
    n_chunks = per_w // chunk
    rows_per_tile = nt // 16
    assert per_w % chunk == 0 and nt % 16 == 0

    @functools.partial(
        pl.kernel,
        out_type=jax.ShapeDtypeStruct((NC, nt, h_dim), jnp.float32),
        mesh=_sc_mesh(),
        compiler_params=pltpu.CompilerParams(use_tc_tiling_on_sc=False),
        scratch_types=[
            pltpu.VMEM((TX_PER_CHUNK, ROWS_PER_TX), jnp.int32),
            pltpu.VMEM((chunk, h_dim), jnp.float32),
            pltpu.VMEM_SHARED((nt, h_dim), jnp.float32),
        ],
    )
    def scatter_add(m_hbm, seg_hbm, zeros_hbm, out_hbm, idx, buf, acc):
        cid = lax.axis_index("c")
        sid = lax.axis_index("s")
        wid = sid * NC + cid
        base_r = wid * (per_w // ROWS_PER_TX)

        @pl.when(sid == 0)
        def _():
            pltpu.sync_copy(zeros_hbm, acc)

        plsc.subcore_barrier()

        @pl.loop(0, n_chunks)
        def _(i):
            r0 = base_r + i * TX_PER_CHUNK
            off = r0 * ROWS_PER_TX
            pltpu.sync_copy(seg_hbm.at[pl.ds(r0, TX_PER_CHUNK)], idx)
            pltpu.sync_copy(m_hbm.at[pl.ds(off, chunk)], buf)
            for j in range(TX_PER_CHUNK):
                pltpu.sync_copy(
                    buf.at[pl.ds(j * ROWS_PER_TX, ROWS_PER_TX)],
                    acc.at[idx.at[j]], add=True)

        plsc.subcore_barrier()
        pltpu.sync_copy(acc.at[pl.ds(sid * rows_per_tile, rows_per_tile)],
                        out_hbm.at[cid].at[pl.ds(sid * rows_per_tile, rows_per_tile)])

    return scatter_add


def _input_net_kernel(x_ref, w1_ref, scale_ref, shift_ref, w2_ref, b2_ref, o_ref):
    h = jnp.dot(x_ref[...], w1_ref[...], preferred_element_type=jnp.float32)
    h = h * scale_ref[...] + shift_ref[...]
    h = jnp.tanh(h)
    h = jnp.dot(h, w2_ref[...], preferred_element_type=jnp.float32) + b2_ref[...]
    o_ref[...] = jnp.maximum(h, 0.0)


def _edge_mlp_kernel(xi_ref, xj_ref, wa_ref, wb_ref, bc1_ref, w2_ref, bc2_ref, o_ref):
    m = jnp.dot(xi_ref[...], wa_ref[...], preferred_element_type=jnp.float32)
    m += jnp.dot(xj_ref[...], wb_ref[...], preferred_element_type=jnp.float32)
    m = jnp.maximum(m + bc1_ref[...], 0.0)
    m = jnp.dot(m, w2_ref[...], preferred_element_type=jnp.float32) + bc2_ref[...]
    o_ref[...] = jnp.maximum(m, 0.0)


def _edge_score_kernel(hs_ref, hd_ref, wa_ref, wb_ref, be1_ref, w2_ref, be2_ref, o_ref):
    e = jnp.dot(hs_ref[...], wa_ref[...], preferred_element_type=jnp.float32)
    e += jnp.dot(hd_ref[...], wb_ref[...], preferred_element_type=jnp.float32)
    e = jnp.maximum(e + be1_ref[...], 0.0)
    e = jnp.dot(e, w2_ref[...], preferred_element_type=jnp.float32) + be2_ref[...]
    o_ref[...] = jax.nn.sigmoid(e)


def _blockdiag(w, copies):
    a, b = w.shape
    eye = jnp.eye(copies, dtype=w.dtype)
    return (eye[:, None, :, None] * w[None, :, None, :]).reshape(copies * a, copies * b)


def _full(shape):
    return pl.BlockSpec(shape, lambda i: tuple(0 for _ in shape))


def _input_net(x, W1, b1, bn_g, bn_b, bn_rm, bn_rv, W2, b2):
    n, d = x.shape
    h_dim = W1.shape[0]
    inv = bn_g / jnp.sqrt(bn_rv + 1e-5)
    scale = inv
    shift = (b1 - bn_rm) * inv + bn_b
    return pl.pallas_call(
        _input_net_kernel,
        grid=(1,),
        in_specs=[_full((n, d)), _full((d, h_dim)), _full((1, h_dim)),
                  _full((1, h_dim)), _full((h_dim, h_dim)), _full((1, h_dim))],
        out_specs=_full((n, h_dim)),
        out_shape=jax.ShapeDtypeStruct((n, h_dim), jnp.float32),
    )(x, W1.T, scale[None], shift[None], W2.T, b2[None])


def _edge_mlp(xi, xj, WA, WB, bc1, W2T, bc2, blk):
    ep, h2 = xi.shape[0], WA.shape[1]
    h_dim = W2T.shape[1]
    return pl.pallas_call(
        _edge_mlp_kernel,
        grid=(ep // blk,),
        in_specs=[pl.BlockSpec((blk, xi.shape[1]), lambda i: (i, 0)),
                  pl.BlockSpec((blk, xj.shape[1]), lambda i: (i, 0)),
                  _full(WA.shape), _full(WB.shape), _full((1, h2)),
                  _full(W2T.shape), _full((1, h_dim))],
        out_specs=pl.BlockSpec((blk, h_dim), lambda i: (i, 0)),
        out_shape=jax.ShapeDtypeStruct((ep, h_dim), jnp.float32),
    )(xi, xj, WA, WB, bc1[None], W2T, bc2[None])


def _edge_score(hs, hd, WA, WB, be1, W2T, be2, blk):
    ep = hs.shape[0]
    h_dim = WA.shape[1]
    return pl.pallas_call(
        _edge_score_kernel,
        grid=(ep // blk,),
        in_specs=[pl.BlockSpec((blk, hs.shape[1]), lambda i: (i, 0)),
                  pl.BlockSpec((blk, hd.shape[1]), lambda i: (i, 0)),
                  _full(WA.shape), _full(WB.shape), _full((1, h_dim)),
                  _full(W2T.shape), _full((1, W2T.shape[1]))],
        out_specs=pl.BlockSpec((blk, W2T.shape[1]), lambda i: (i, 0)),
        out_shape=jax.ShapeDtypeStruct((ep, W2T.shape[1]), jnp.float32),
    )(hs, hd, WA, WB, be1[None], W2T, be2[None])


def _pad_to(a, size, fill):
    return jnp.concatenate([a, jnp.full((size - a.shape[0],), fill, a.dtype)])


def kernel(x, edge_index, W1, b1, bn_g, bn_b, bn_rm, bn_rv, W2, b2, Wc1, bc1, Wc2, bc2, We1, be1, We2, be2):
    n = x.shape[0]
    e_cnt = edge_index.shape[1]
    h_dim = W1.shape[0]
    nt = n + 16

    loops = jnp.arange(n, dtype=edge_index.dtype)
    ei = jnp.concatenate([edge_index, jnp.stack([loops, loops])], axis=1)
    ei2 = jnp.concatenate([ei, ei[::-1]], axis=1)
    k = ei2[0] * n + ei2[1]
    ks = jnp.sort(k)
    mask = jnp.concatenate([jnp.ones((1,), dtype=bool), ks[1:] != ks[:-1]])
    row = ks // n
    col = ks % n

    ec = ks.shape[0]
    grain = NW * ROWS_PER_TX * TX_PER_CHUNK * 2
    ec_pad = ((ec + grain - 1) // grain) * grain
    rowp = _pad_to(row, ec_pad, 0)
    colp = _pad_to(col, ec_pad, 0)
    segp = _pad_to(jnp.where(mask, row, n), ec_pad, n).reshape(
        ec_pad // ROWS_PER_TX, ROWS_PER_TX)

    h = _input_net(x, W1, b1, bn_g, bn_b, bn_rm, bn_rv, W2, b2)

    A = Wc1[:, :h_dim]
    B = Wc1[:, h_dim:]
    WA4 = _blockdiag((A - B).T, 4)
    WB4 = _blockdiag(B.T, 4)
    W2T4 = _blockdiag(Wc2.T, 4)
    bc1t = jnp.tile(bc1, 4)
    bc2t = jnp.tile(bc2, 4)

    gather2 = _make_gather2(n, h_dim, ec_pad)
    scatter_add = _make_scatter_add(nt, h_dim, ec_pad)
    zeros_nt = jnp.zeros((nt, h_dim), jnp.float32)
    blk4 = 2048

    for _ in range(3):
        xi, xj = gather2(h, rowp, colp)
        xi4 = xi.reshape(ec_pad // 4, 4 * h_dim)
        xj4 = xj.reshape(ec_pad // 4, 4 * h_dim)
        m4 = _edge_mlp(xi4, xj4, WA4, WB4, bc1t, W2T4, bc2t, blk4)
        m = m4.reshape(ec_pad, h_dim)
        partials = scatter_add(m, segp, zeros_nt)
        h = partials[0, :n] + partials[1, :n]

    e_pad = ((e_cnt + grain - 1) // grain) * grain
    srcp = _pad_to(edge_index[0], e_pad, 0)
    dstp = _pad_to(edge_index[1], e_pad, 0)
    hs, hd = gather2(h, srcp, dstp) if e_pad == ec_pad else _make_gather2(n, h_dim, e_pad)(h, srcp, dstp)
    EA4 = _blockdiag(We1[:, :h_dim].T, 4)
    EB4 = _blockdiag(We1[:, h_dim:].T, 4)
    W2c4 = _blockdiag(We2.T, 4)
    be1t = jnp.tile(be1, 4)
    be2t = jnp.tile(be2, 4)
    hs4 = hs.reshape(e_pad // 4, 4 * h_dim)
    hd4 = hd.reshape(e_pad // 4, 4 * h_dim)
    scores4 = _edge_score(hs4, hd4, EA4, EB4, be1t, W2c4, be2t, blk4)
    return scores4.reshape(e_pad)[:e_cnt]


# speedup vs baseline: 4.7110x; 1.1339x over previous
import functools

import jax
import jax.numpy as jnp
from jax import lax
from jax.experimental import pallas as pl
from jax.experimental.pallas import tpu as pltpu
from jax.experimental.pallas import tpu_sc as plsc

NW = 32
NC = 2
ROWS_PER_TX = 128
TX_PER_CHUNK = 4


def _sc_mesh():
    return plsc.VectorSubcoreMesh(core_axis_name="c", subcore_axis_name="s")


def _make_gather2(n_pad, h_dim, ec_pad):
    chunk = ROWS_PER_TX * TX_PER_CHUNK
    per_w = ec_pad // NW
    n_chunks = per_w // chunk
    assert per_w % chunk == 0 and n_chunks % 2 == 0

    @functools.partial(
        pl.kernel,
        out_type=[jax.ShapeDtypeStruct((ec_pad, h_dim), jnp.float32),
                  jax.ShapeDtypeStruct((ec_pad, h_dim), jnp.float32)],
        mesh=_sc_mesh(),
        compiler_params=pltpu.CompilerParams(use_tc_tiling_on_sc=False),
        scratch_types=[
            [pltpu.VMEM((chunk,), jnp.int32) for _ in range(2)],
            [pltpu.VMEM((chunk,), jnp.int32) for _ in range(2)],
            [pltpu.VMEM((chunk, h_dim), jnp.float32) for _ in range(2)],
            [pltpu.VMEM((chunk, h_dim), jnp.float32) for _ in range(2)],
            [pltpu.SemaphoreType.DMA for _ in range(2)],
            [pltpu.SemaphoreType.DMA for _ in range(2)],
            [pltpu.SemaphoreType.DMA for _ in range(2)],
        ],
    )
    def gather2(tab_hbm, rows_hbm, cols_hbm, xi_hbm, xj_hbm, ia, ib, ba, bb,
                sidx, sg, sf):
        wid = lax.axis_index("s") * NC + lax.axis_index("c")
        base = wid * per_w

        def idx_copies(ci, b):
            off = base + ci * chunk
            return (pltpu.make_async_copy(rows_hbm.at[pl.ds(off, chunk)], ia[b], sidx[b]),
                    pltpu.make_async_copy(cols_hbm.at[pl.ds(off, chunk)], ib[b], sidx[b]))

        def gather_copies(b):
            return (pltpu.make_async_copy(tab_hbm.at[ia[b]], ba[b], sg[b]),
                    pltpu.make_async_copy(tab_hbm.at[ib[b]], bb[b], sg[b]))

        def flush_copies(ci, b):
            off = base + ci * chunk
            return (pltpu.make_async_copy(ba[b], xi_hbm.at[pl.ds(off, chunk)], sf[b]),
                    pltpu.make_async_copy(bb[b], xj_hbm.at[pl.ds(off, chunk)], sf[b]))

        for cp in idx_copies(0, 0):
            cp.start()

        @pl.loop(0, n_chunks, step=2)
        def _(i):
            for b in range(2):
                ci = i + b
                for cp in idx_copies(ci, b):
                    cp.wait()

                @pl.when(ci >= 2)
                def _():
                    for cp in flush_copies(ci, b):
                        cp.wait()

                for cp in gather_copies(b):
                    cp.start()

                @pl.when(ci >= 1)
                def _():
                    for cp in gather_copies(1 - b):
                        cp.wait()
                    for cp in flush_copies(ci - 1, 1 - b):
                        cp.start()

                @pl.when(ci + 1 < n_chunks)
                def _():
                    for cp in idx_copies(ci + 1, 1 - b):
                        cp.start()

        last = n_chunks - 1
        for cp in gather_copies(last % 2):
            cp.wait()
        for cp in flush_copies(last, last % 2):
            cp.start()
        for b in range(2):
            for cp in flush_copies(n_chunks - 2 + b, b):
                cp.wait()

    return gather2


def _make_scatter_add(nt, h_dim, ec_pad):
    chunk = ROWS_PER_TX * TX_PER_CHUNK
    per_w = ec_pad // NW
    n_chunks = per_w // chunk
    rows_per_tile = nt // 16
    assert per_w % chunk == 0 and nt % 16 == 0

    @functools.partial(
        pl.kernel,
        out_type=jax.ShapeDtypeStruct((NC, nt, h_dim), jnp.float32),
        mesh=_sc_mesh(),
        compiler_params=pltpu.CompilerParams(use_tc_tiling_on_sc=False),
        scratch_types=[
            pltpu.VMEM((TX_PER_CHUNK, ROWS_PER_TX), jnp.int32),
            pltpu.VMEM((chunk, h_dim), jnp.float32),
            pltpu.VMEM_SHARED((nt, h_dim), jnp.float32),
        ],
    )
    def scatter_add(m_hbm, seg_hbm, zeros_hbm, out_hbm, idx, buf, acc):
        cid = lax.axis_index("c")
        sid = lax.axis_index("s")
        wid = sid * NC + cid
        base_r = wid * (per_w // ROWS_PER_TX)

        @pl.when(sid == 0)
        def _():
            pltpu.sync_copy(zeros_hbm, acc)

        plsc.subcore_barrier()

        @pl.loop(0, n_chunks)
        def _(i):
            r0 = base_r + i * TX_PER_CHUNK
            off = r0 * ROWS_PER_TX
            pltpu.sync_copy(seg_hbm.at[pl.ds(r0, TX_PER_CHUNK)], idx)
            pltpu.sync_copy(m_hbm.at[pl.ds(off, chunk)], buf)
            for j in range(TX_PER_CHUNK):
                pltpu.sync_copy(
                    buf.at[pl.ds(j * ROWS_PER_TX, ROWS_PER_TX)],
                    acc.at[idx.at[j]], add=True)

        plsc.subcore_barrier()
        pltpu.sync_copy(acc.at[pl.ds(sid * rows_per_tile, rows_per_tile)],
                        out_hbm.at[cid].at[pl.ds(sid * rows_per_tile, rows_per_tile)])

    return scatter_add


def _input_net_kernel(x_ref, w1_ref, scale_ref, shift_ref, w2_ref, b2_ref, o_ref):
    h = jnp.dot(x_ref[...], w1_ref[...], preferred_element_type=jnp.float32)
    h = h * scale_ref[...] + shift_ref[...]
    h = jnp.tanh(h)
    h = jnp.dot(h, w2_ref[...], preferred_element_type=jnp.float32) + b2_ref[...]
    o_ref[...] = jnp.maximum(h, 0.0)


def _edge_mlp_kernel(xi_ref, xj_ref, wa_ref, wb_ref, bc1_ref, w2_ref, bc2_ref, o_ref):
    m = jnp.dot(xi_ref[...], wa_ref[...], preferred_element_type=jnp.float32)
    m += jnp.dot(xj_ref[...], wb_ref[...], preferred_element_type=jnp.float32)
    m = jnp.maximum(m + bc1_ref[...], 0.0)
    m = jnp.dot(m, w2_ref[...], preferred_element_type=jnp.float32) + bc2_ref[...]
    o_ref[...] = jnp.maximum(m, 0.0)


def _edge_score_kernel(hs_ref, hd_ref, wa_ref, wb_ref, be1_ref, w2_ref, be2_ref, o_ref):
    e = jnp.dot(hs_ref[...], wa_ref[...], preferred_element_type=jnp.float32)
    e += jnp.dot(hd_ref[...], wb_ref[...], preferred_element_type=jnp.float32)
    e = jnp.maximum(e + be1_ref[...], 0.0)
    e = jnp.dot(e, w2_ref[...], preferred_element_type=jnp.float32) + be2_ref[...]
    o_ref[...] = jax.nn.sigmoid(e)


def _blockdiag(w, copies):
    a, b = w.shape
    eye = jnp.eye(copies, dtype=w.dtype)
    return (eye[:, None, :, None] * w[None, :, None, :]).reshape(copies * a, copies * b)


def _full(shape):
    return pl.BlockSpec(shape, lambda i: tuple(0 for _ in shape))


def _input_net(x, W1, b1, bn_g, bn_b, bn_rm, bn_rv, W2, b2):
    n, d = x.shape
    h_dim = W1.shape[0]
    inv = bn_g / jnp.sqrt(bn_rv + 1e-5)
    scale = inv
    shift = (b1 - bn_rm) * inv + bn_b
    return pl.pallas_call(
        _input_net_kernel,
        grid=(1,),
        in_specs=[_full((n, d)), _full((d, h_dim)), _full((1, h_dim)),
                  _full((1, h_dim)), _full((h_dim, h_dim)), _full((1, h_dim))],
        out_specs=_full((n, h_dim)),
        out_shape=jax.ShapeDtypeStruct((n, h_dim), jnp.float32),
    )(x, W1.T, scale[None], shift[None], W2.T, b2[None])


def _edge_mlp(xi, xj, WA, WB, bc1, W2T, bc2, blk):
    ep, h2 = xi.shape[0], WA.shape[1]
    h_dim = W2T.shape[1]
    return pl.pallas_call(
        _edge_mlp_kernel,
        grid=(ep // blk,),
        in_specs=[pl.BlockSpec((blk, xi.shape[1]), lambda i: (i, 0)),
                  pl.BlockSpec((blk, xj.shape[1]), lambda i: (i, 0)),
                  _full(WA.shape), _full(WB.shape), _full((1, h2)),
                  _full(W2T.shape), _full((1, h_dim))],
        out_specs=pl.BlockSpec((blk, h_dim), lambda i: (i, 0)),
        out_shape=jax.ShapeDtypeStruct((ep, h_dim), jnp.float32),
    )(xi, xj, WA, WB, bc1[None], W2T, bc2[None])


def _edge_score(hs, hd, WA, WB, be1, W2T, be2, blk):
    ep = hs.shape[0]
    h_dim = WA.shape[1]
    return pl.pallas_call(
        _edge_score_kernel,
        grid=(ep // blk,),
        in_specs=[pl.BlockSpec((blk, hs.shape[1]), lambda i: (i, 0)),
                  pl.BlockSpec((blk, hd.shape[1]), lambda i: (i, 0)),
                  _full(WA.shape), _full(WB.shape), _full((1, h_dim)),
                  _full(W2T.shape), _full((1, W2T.shape[1]))],
        out_specs=pl.BlockSpec((blk, W2T.shape[1]), lambda i: (i, 0)),
        out_shape=jax.ShapeDtypeStruct((ep, W2T.shape[1]), jnp.float32),
    )(hs, hd, WA, WB, be1[None], W2T, be2[None])


def _pad_to(a, size, fill):
    return jnp.concatenate([a, jnp.full((size - a.shape[0],), fill, a.dtype)])


def kernel(x, edge_index, W1, b1, bn_g, bn_b, bn_rm, bn_rv, W2, b2, Wc1, bc1, Wc2, bc2, We1, be1, We2, be2):
    n = x.shape[0]
    e_cnt = edge_index.shape[1]
    h_dim = W1.shape[0]
    nt = n + 16

    loops = jnp.arange(n, dtype=edge_index.dtype)
    ei = jnp.concatenate([edge_index, jnp.stack([loops, loops])], axis=1)
    ei2 = jnp.concatenate([ei, ei[::-1]], axis=1)
    k = ei2[0] * n + ei2[1]
    ks = jnp.sort(k)
    mask = jnp.concatenate([jnp.ones((1,), dtype=bool), ks[1:] != ks[:-1]])
    row = ks // n
    col = ks % n

    ec = ks.shape[0]
    grain = NW * ROWS_PER_TX * TX_PER_CHUNK * 2
    ec_pad = ((ec + grain - 1) // grain) * grain
    rowp = _pad_to(row, ec_pad, 0)
    colp = _pad_to(col, ec_pad, 0)
    segp = _pad_to(jnp.where(mask, row, n), ec_pad, n).reshape(
        ec_pad // ROWS_PER_TX, ROWS_PER_TX)

    h = _input_net(x, W1, b1, bn_g, bn_b, bn_rm, bn_rv, W2, b2)

    A = Wc1[:, :h_dim]
    B = Wc1[:, h_dim:]
    WA4 = _blockdiag((A - B).T, 4)
    WB4 = _blockdiag(B.T, 4)
    W2T4 = _blockdiag(Wc2.T, 4)
    bc1t = jnp.tile(bc1, 4)
    bc2t = jnp.tile(bc2, 4)

    gather2 = _make_gather2(n, h_dim, ec_pad)
    scatter_add = _make_scatter_add(nt, h_dim, ec_pad)
    zeros_nt = jnp.zeros((nt, h_dim), jnp.float32)
    blk4 = 2048

    REP = 4
    per_w = ec_pad // NW
    off_ec = ((jnp.arange(ec_pad, dtype=jnp.int32) // per_w) % REP) * n
    rowp = rowp + off_ec
    colp = colp + off_ec

    for _ in range(3):
        hrep = jnp.tile(h, (REP, 1))
        xi, xj = gather2(hrep, rowp, colp)
        xi4 = xi.reshape(ec_pad // 4, 4 * h_dim)
        xj4 = xj.reshape(ec_pad // 4, 4 * h_dim)
        m4 = _edge_mlp(xi4, xj4, WA4, WB4, bc1t, W2T4, bc2t, blk4)
        m = m4.reshape(ec_pad, h_dim)
        partials = scatter_add(m, segp, zeros_nt)
        h = partials[0, :n] + partials[1, :n]

    e_pad = ((e_cnt + grain - 1) // grain) * grain
    off_e = ((jnp.arange(e_pad, dtype=jnp.int32) // (e_pad // NW)) % REP) * n
    srcp = _pad_to(edge_index[0], e_pad, 0) + off_e
    dstp = _pad_to(edge_index[1], e_pad, 0) + off_e
    hrep = jnp.tile(h, (REP, 1))
    hs, hd = gather2(hrep, srcp, dstp) if e_pad == ec_pad else _make_gather2(n, h_dim, e_pad)(hrep, srcp, dstp)
    EA4 = _blockdiag(We1[:, :h_dim].T, 4)
    EB4 = _blockdiag(We1[:, h_dim:].T, 4)
    W2c4 = _blockdiag(We2.T, 4)
    be1t = jnp.tile(be1, 4)
    be2t = jnp.tile(be2, 4)
    hs4 = hs.reshape(e_pad // 4, 4 * h_dim)
    hd4 = hd.reshape(e_pad // 4, 4 * h_dim)
    scores4 = _edge_score(hs4, hd4, EA4, EB4, be1t, W2c4, be2t, blk4)
    return scores4.reshape(e_pad)[:e_cnt]
